# Initial kernel scaffold; baseline (speedup 1.0000x reference)
import jax
import jax.numpy as jnp
from jax.experimental import pallas as pl


def kernel(startends, attention, context):
    raise NotImplementedError("write your pallas kernel here")


# speedup vs baseline: 2.7065x; 2.7065x over previous
import jax
import jax.numpy as jnp
from jax.experimental import pallas as pl
from jax.experimental.pallas import tpu as pltpu

_BATCH = 16
_N = 32
_S = 2048
_L = 2048
_D = 768
_C = 256
_NCH = _L // _C


def _kern(se_ref, att_ref, ctx_ref, out_ref):
    att = att_ref[0, :, :]
    starts = se_ref[0, :, 0].reshape(_N, 1)
    ends = se_ref[0, :, 1].reshape(_N, 1)
    pos = jax.lax.broadcasted_iota(jnp.int32, (_N, _S), 1)
    m = (pos >= starts) & (pos < ends)
    sums = jnp.sum(jnp.where(m, att, 0.0), axis=1, keepdims=True)
    maxv = jnp.max(sums)
    idx = jax.lax.broadcasted_iota(jnp.int32, (_N, 1), 0)
    best = jnp.min(jnp.where(sums == maxv, idx, _N))
    sel = maxv > 0.0
    is_best = idx == best
    start = jnp.where(sel, jnp.sum(jnp.where(is_best, starts, 0)), 0)
    end = jnp.where(sel, jnp.sum(jnp.where(is_best, ends, 0)), 0)
    nv = end - start

    _W = _C + 8
    for c in range(_NCH):
        lo = c * _C

        @pl.when(nv <= lo)
        def _():
            out_ref[0, lo:lo + _C, :] = jnp.zeros((_C, _D), jnp.float32)

        @pl.when(nv > lo)
        def _():
            roff = jnp.minimum((start + lo) // 8 * 8, _S - _W)
            roff = pl.multiple_of(roff, 8)
            t = start + lo - roff
            win = ctx_ref[0, pl.ds(roff, _W), :]
            shift = jax.lax.rem(_W - t, _W)
            rot = pltpu.roll(win, shift, axis=0)
            rows = jax.lax.broadcasted_iota(jnp.int32, (_C, 1), 0)
            valid = rows < (nv - lo)
            out_ref[0, lo:lo + _C, :] = jnp.where(
                valid, rot[0:_C, :], 0.0)


@jax.jit
def kernel(startends, attention, context):
    att3 = attention.reshape(_BATCH, 1, _S)
    return pl.pallas_call(
        _kern,
        grid=(_BATCH,),
        in_specs=[
            pl.BlockSpec((1, _N, 2), lambda b: (b, 0, 0)),
            pl.BlockSpec((1, 1, _S), lambda b: (b, 0, 0)),
            pl.BlockSpec((1, _S, _D), lambda b: (b, 0, 0)),
        ],
        out_specs=pl.BlockSpec((1, _L, _D), lambda b: (b, 0, 0)),
        out_shape=jax.ShapeDtypeStruct((_BATCH, _L, _D), jnp.float32),
        compiler_params=pltpu.CompilerParams(
            dimension_semantics=("arbitrary",)),
    )(startends, att3, context)
